# baseline (speedup 1.0000x reference)
import functools

import jax
import jax.numpy as jnp
from jax import lax
from jax.experimental import pallas as pl
from jax.experimental.pallas import tpu as pltpu
from jax.experimental.pallas import tpu_sc as plsc

NUM_BOXES = 100000
DIM = 64
BATCH = 16384
ROW = 2 * DIM

NC = 2
NS = 16
NW = NC * NS
B_PER_W = BATCH // NW
CHUNK = 128
NCHUNK = B_PER_W // CHUNK
GROUPS = CHUNK // 16

_LN2 = 0.6931471805599453
_SQRT2 = 1.4142135623730951


def _poslog(y):
  bits = plsc.bitcast(y, jnp.int32)
  e = lax.shift_right_arithmetic(bits, 23) - 127
  mbits = lax.bitwise_or(lax.bitwise_and(bits, 0x007FFFFF), 0x3F800000)
  m = plsc.bitcast(mbits, jnp.float32)
  big = m > _SQRT2
  m = jnp.where(big, m * 0.5, m)
  ef = e.astype(jnp.float32) + jnp.where(big, 1.0, 0.0)
  s = (m - 1.0) / (m + 1.0)
  z = s * s
  p = 2.0 * s * (1.0 + z * (1.0 / 3.0 + z * (1.0 / 5.0 + z * (1.0 / 7.0))))
  return ef * _LN2 + p


def _softplus(x):
  return _poslog(1.0 + jnp.exp(x))


def _sc_body(tab, idx1, idx2, out,
             idx1_v, idx2_v, rows1a, rows1b, rows2a, rows2b, out_v,
             sem_a, sem_b):
  wid = lax.axis_index("s") * NC + lax.axis_index("c")

  pltpu.sync_copy(idx1.at[wid], idx1_v)
  pltpu.sync_copy(idx2.at[wid], idx2_v)

  rows1 = (rows1a, rows1b)
  rows2 = (rows2a, rows2b)
  sems = (sem_a, sem_b)

  def start(c):
    slot = c % 2
    pltpu.make_async_copy(tab.at[idx1_v.at[c]], rows1[slot], sems[slot]).start()
    pltpu.make_async_copy(tab.at[idx2_v.at[c]], rows2[slot], sems[slot]).start()

  def wait(c):
    slot = c % 2
    pltpu.make_async_copy(tab.at[idx1_v.at[c]], rows1[slot], sems[slot]).wait()
    pltpu.make_async_copy(tab.at[idx2_v.at[c]], rows2[slot], sems[slot]).wait()

  start(0)
  for c in range(NCHUNK):
    wait(c)
    if c + 1 < NCHUNK:
      start(c + 1)
    r1 = rows1[c % 2]
    r2 = rows2[c % 2]

    def group_body(g, _, r1=r1, r2=r2, c=c):
      rlane = g * 16 + lax.iota(jnp.int32, 16)

      def dim_body(d, carry):
        pi, p2 = carry
        cmin = jnp.full((16,), 0, jnp.int32) + d
        cdel = cmin + DIM
        m1 = plsc.load_gather(r1, [rlane, cmin])
        e1 = plsc.load_gather(r1, [rlane, cdel])
        m2 = plsc.load_gather(r2, [rlane, cmin])
        e2 = plsc.load_gather(r2, [rlane, cdel])
        x1 = jnp.exp(e1)
        x2 = jnp.exp(e2)
        imin = jnp.maximum(m1, m2)
        imax = jnp.minimum(m1 + x1, m2 + x2)
        t = jnp.clip(imax - imin, 1e-7, 10000.0)
        return pi * _softplus(t), p2 * _softplus(x2)

      ones = jnp.full((16,), 1.0, jnp.float32)
      pi, p2 = lax.fori_loop(0, DIM, dim_body, (ones, ones))
      res = jnp.maximum(pi / p2, 0.0)
      out_v[pl.ds(c * CHUNK + g * 16, 16)] = res
      return 0

    lax.fori_loop(0, GROUPS, group_body, 0)

  pltpu.sync_copy(out_v, out.at[pl.ds(wid * B_PER_W, B_PER_W)])


@functools.partial(
    pl.kernel,
    out_type=jax.ShapeDtypeStruct((BATCH,), jnp.float32),
    mesh=plsc.VectorSubcoreMesh(
        core_axis_name="c", subcore_axis_name="s",
        num_cores=NC, num_subcores=NS),
    scratch_types=[
        pltpu.VMEM((NCHUNK, CHUNK), jnp.int32),
        pltpu.VMEM((NCHUNK, CHUNK), jnp.int32),
        pltpu.VMEM((CHUNK, ROW), jnp.float32),
        pltpu.VMEM((CHUNK, ROW), jnp.float32),
        pltpu.VMEM((CHUNK, ROW), jnp.float32),
        pltpu.VMEM((CHUNK, ROW), jnp.float32),
        pltpu.VMEM((B_PER_W,), jnp.float32),
        pltpu.SemaphoreType.DMA,
        pltpu.SemaphoreType.DMA,
    ],
    compiler_params=pltpu.CompilerParams(needs_layout_passes=False),
)
def _boxes_sc(*refs):
  _sc_body(*refs)


def kernel(X, boxes):
  tab = boxes.reshape(NUM_BOXES, ROW)
  xi = X.astype(jnp.int32)
  idx1 = xi[:, 0].reshape(NW, NCHUNK, CHUNK)
  idx2 = xi[:, 1].reshape(NW, NCHUNK, CHUNK)
  return _boxes_sc(tab, idx1, idx2)
